# baseline (speedup 1.0000x reference)
import functools

import jax
import jax.numpy as jnp
from jax import lax
from jax.experimental import pallas as pl
from jax.experimental.pallas import tpu as pltpu
from jax.experimental.pallas import tpu_sc as plsc

_LANES = 16


def _build_sc_call(R, D, T, A, P, AP):
    NC, NS = 2, 16
    NW = NC * NS
    CH = 50
    CW = R // (NW * CH)
    CPA = (P * T) // CH
    NV = D // _LANES

    assert R == NW * CH * CW and T % CH == 0 and (P * T) % CH == 0

    mesh = plsc.VectorSubcoreMesh(core_axis_name="c", subcore_axis_name="s")

    @functools.partial(
        pl.kernel,
        out_type=jax.ShapeDtypeStruct((R, D), jnp.float32),
        mesh=mesh,
        compiler_params=pltpu.CompilerParams(use_tc_tiling_on_sc=False),
        scratch_types=[
            pltpu.VMEM((T, D), jnp.float32),
            pltpu.VMEM((AP, D), jnp.float32),
            pltpu.VMEM((AP,), jnp.int32),
            pltpu.VMEM((CH, D), jnp.float32),
            pltpu.VMEM((CH, D), jnp.float32),
            pltpu.VMEM((CH, D), jnp.float32),
            pltpu.VMEM((CH, D), jnp.float32),
            pltpu.SemaphoreType.DMA,
            pltpu.SemaphoreType.DMA,
            pltpu.SemaphoreType.DMA,
            pltpu.SemaphoreType.DMA,
            pltpu.SemaphoreType.DMA,
        ],
    )
    def sc_add(x_hbm, pe_hbm, ie_hbm, idx_hbm, out_hbm,
               pe_b, ie_b, idx_b, ib0, ib1, ob0, ob1,
               si0, si1, so0, so1, sg):
        wid = lax.axis_index("s") * NC + lax.axis_index("c")
        base = wid * (CW * CH)

        pltpu.sync_copy(idx_hbm, idx_b)
        pltpu.sync_copy(pe_hbm, pe_b)
        pltpu.async_copy(ie_hbm.at[idx_b], ie_b, sg).wait()

        bufs = ((ib0, ob0, si0, so0), (ib1, ob1, si1, so1))

        def start_in(k, ib, si):
            pltpu.async_copy(x_hbm.at[pl.ds(base + k * CH, CH)], ib, si)

        def start_out(k, ob, so):
            pltpu.async_copy(ob, out_hbm.at[pl.ds(base + k * CH, CH)], so)

        def wait_in(ib, si):
            pltpu.make_async_copy(x_hbm.at[pl.ds(0, CH)], ib, si).wait()

        def wait_out(ob, so):
            pltpu.make_async_copy(ob, out_hbm.at[pl.ds(0, CH)], so).wait()

        start_in(0, ib0, si0)
        start_in(1, ib1, si1)

        gi0 = wid * CW
        a0 = jnp.right_shift(5 * wid, 3)
        ac0 = gi0 - a0 * CPA
        poff0 = jnp.bitwise_and(wid, 1) * (2 * CH)

        def compute(ib, ob, a, poff):
            ie_vecs = [ie_b[a, pl.ds(j * _LANES, _LANES)] for j in range(NV)]

            def row(r, c):
                for j in range(NV):
                    sl = pl.ds(j * _LANES, _LANES)
                    ob[r, sl] = ib[r, sl] + pe_b[poff + r, sl] + ie_vecs[j]
                return c

            lax.fori_loop(0, CH, row, 0)

        def pair(p, carry):
            a, ac, poff = carry
            for b in range(2):
                ib, ob, si, so = bufs[b]
                k = 2 * p + b
                wait_in(ib, si)

                @pl.when(p > 0)
                def _():
                    wait_out(ob, so)

                compute(ib, ob, a, poff)
                start_out(k, ob, so)

                @pl.when(k + 2 < CW)
                def _():
                    start_in(k + 2, ib, si)

                poff = jnp.where(poff + CH == T, 0, poff + CH)
                ac = ac + 1
                bump = ac == CPA
                a = jnp.where(bump, a + 1, a)
                ac = jnp.where(bump, 0, ac)
            return (a, ac, poff)

        lax.fori_loop(0, CW // 2, pair,
                      (a0, ac0.astype(jnp.int32), poff0.astype(jnp.int32)))

        wait_out(ob0, so0)
        wait_out(ob1, so1)

    return sc_add


def kernel(x, pe, ie, id_enc_shuffle, num_a, num_p, num_t, t_offset):
    B, N, D = x.shape
    A = id_enc_shuffle.shape[0]
    T = pe.shape[1]
    P = N // (A * T)
    AP = 32

    x2 = x.reshape(N, D)
    pe2 = pe.reshape(T, D)
    ie2 = ie.reshape(ie.shape[1], D)
    idxp = jnp.zeros((AP,), jnp.int32).at[:A].set(id_enc_shuffle.astype(jnp.int32))

    out2 = _build_sc_call(N, D, T, A, P, AP)(x2, pe2, ie2, idxp)
    return out2.reshape(B, N, D)


# speedup vs baseline: 1.7355x; 1.7355x over previous
import functools

import jax
import jax.numpy as jnp
from jax import lax
from jax.experimental import pallas as pl
from jax.experimental.pallas import tpu as pltpu
from jax.experimental.pallas import tpu_sc as plsc

_LANES = 16


def _build_sc_call(R, D, T, A, P, AP):
    NC, NS = 2, 16
    NW = NC * NS
    CH = 50
    CW = R // (NW * CH)
    CPA = (P * T) // CH
    NV = D // _LANES

    assert R == NW * CH * CW and T % CH == 0 and (P * T) % CH == 0

    mesh = plsc.VectorSubcoreMesh(core_axis_name="c", subcore_axis_name="s")

    @functools.partial(
        pl.kernel,
        out_type=jax.ShapeDtypeStruct((R, D), jnp.float32),
        mesh=mesh,
        compiler_params=pltpu.CompilerParams(use_tc_tiling_on_sc=False),
        scratch_types=[
            pltpu.VMEM((T, D), jnp.float32),
            pltpu.VMEM((AP, D), jnp.float32),
            pltpu.VMEM((AP,), jnp.int32),
            pltpu.VMEM((CH, D), jnp.float32),
            pltpu.VMEM((CH, D), jnp.float32),
            pltpu.VMEM((CH, D), jnp.float32),
            pltpu.VMEM((CH, D), jnp.float32),
            pltpu.SemaphoreType.DMA,
            pltpu.SemaphoreType.DMA,
            pltpu.SemaphoreType.DMA,
            pltpu.SemaphoreType.DMA,
            pltpu.SemaphoreType.DMA,
        ],
    )
    def sc_add(x_hbm, pe_hbm, ie_hbm, idx_hbm, out_hbm,
               pe_b, ie_b, idx_b, ib0, ib1, ob0, ob1,
               si0, si1, so0, so1, sg):
        wid = lax.axis_index("s") * NC + lax.axis_index("c")
        base = wid * (CW * CH)

        pltpu.sync_copy(idx_hbm, idx_b)
        pltpu.sync_copy(pe_hbm, pe_b)
        pltpu.async_copy(ie_hbm.at[idx_b], ie_b, sg).wait()

        bufs = ((ib0, ob0, si0, so0), (ib1, ob1, si1, so1))

        def start_in(k, ib, si):
            pltpu.async_copy(x_hbm.at[pl.ds(base + k * CH, CH)], ib, si)

        def start_out(k, ob, so):
            pltpu.async_copy(ob, out_hbm.at[pl.ds(base + k * CH, CH)], so)

        def wait_in(ib, si):
            pltpu.make_async_copy(x_hbm.at[pl.ds(0, CH)], ib, si).wait()

        def wait_out(ob, so):
            pltpu.make_async_copy(ob, out_hbm.at[pl.ds(0, CH)], so).wait()

        start_in(0, ib0, si0)
        start_in(1, ib1, si1)

        gi0 = wid * CW
        a0 = jnp.right_shift(5 * wid, 3)
        ac0 = gi0 - a0 * CPA
        poff0 = jnp.bitwise_and(wid, 1) * (2 * CH)

        def compute(ib, ob, a, poff):
            ie_vecs = [ie_b[a, pl.ds(j * _LANES, _LANES)] for j in range(NV)]

            @plsc.parallel_loop(0, CH, unroll=2)
            def row(r):
                for j in range(NV):
                    sl = pl.ds(j * _LANES, _LANES)
                    ob[r, sl] = ib[r, sl] + pe_b[poff + r, sl] + ie_vecs[j]

        def pair(p, carry):
            a, ac, poff = carry
            for b in range(2):
                ib, ob, si, so = bufs[b]
                k = 2 * p + b
                wait_in(ib, si)

                @pl.when(p > 0)
                def _():
                    wait_out(ob, so)

                compute(ib, ob, a, poff)
                start_out(k, ob, so)

                @pl.when(k + 2 < CW)
                def _():
                    start_in(k + 2, ib, si)

                poff = jnp.where(poff + CH == T, 0, poff + CH)
                ac = ac + 1
                bump = ac == CPA
                a = jnp.where(bump, a + 1, a)
                ac = jnp.where(bump, 0, ac)
            return (a, ac, poff)

        lax.fori_loop(0, CW // 2, pair,
                      (a0, ac0.astype(jnp.int32), poff0.astype(jnp.int32)))

        wait_out(ob0, so0)
        wait_out(ob1, so1)

    return sc_add


def kernel(x, pe, ie, id_enc_shuffle, num_a, num_p, num_t, t_offset):
    B, N, D = x.shape
    A = id_enc_shuffle.shape[0]
    T = pe.shape[1]
    P = N // (A * T)
    AP = 32

    x2 = x.reshape(N, D)
    pe2 = pe.reshape(T, D)
    ie2 = ie.reshape(ie.shape[1], D)
    idxp = jnp.zeros((AP,), jnp.int32).at[:A].set(id_enc_shuffle.astype(jnp.int32))

    out2 = _build_sc_call(N, D, T, A, P, AP)(x2, pe2, ie2, idxp)
    return out2.reshape(B, N, D)


# speedup vs baseline: 4.2198x; 2.4315x over previous
import functools

import jax
import jax.numpy as jnp
import numpy as np
from jax import lax
from jax.experimental import pallas as pl
from jax.experimental.pallas import tpu as pltpu
from jax.experimental.pallas import tpu_sc as plsc

_LANES = 16


def _build_sc_call(R, D, T, A, P, AP):
    NC, NS = 2, 16
    NW = NC * NS
    CH = 40
    CPB = T // CH
    CPA = (P * T) // CH
    NV = D // _LANES

    NBLK = R // T
    assert T % CH == 0 and (P * T) % CH == 0 and CH % 8 == 0

    mesh = plsc.VectorSubcoreMesh(core_axis_name="c", subcore_axis_name="s")

    @functools.partial(
        pl.kernel,
        out_type=jax.ShapeDtypeStruct((R, D), jnp.float32),
        mesh=mesh,
        scratch_types=[
            pltpu.VMEM((T, D), jnp.float32),
            pltpu.VMEM((AP, D), jnp.float32),
            pltpu.VMEM((AP, _LANES), jnp.int32),
            pltpu.VMEM((NW, _LANES), jnp.int32),
            pltpu.VMEM((CH, D), jnp.float32),
            pltpu.VMEM((CH, D), jnp.float32),
            pltpu.VMEM((CH, D), jnp.float32),
            pltpu.VMEM((CH, D), jnp.float32),
            pltpu.SemaphoreType.DMA,
            pltpu.SemaphoreType.DMA,
            pltpu.SemaphoreType.DMA,
            pltpu.SemaphoreType.DMA,
        ],
    )
    def sc_add(x_hbm, pe_hbm, ie_hbm, idx_hbm, winit_hbm, out_hbm,
               pe_b, ie_b, idx_b, wi_b, ib0, ib1, ob0, ob1,
               si0, si1, so0, so1):
        wid = lax.axis_index("s") * NC + lax.axis_index("c")

        pltpu.sync_copy(winit_hbm, wi_b)
        pltpu.sync_copy(idx_hbm, idx_b)
        pltpu.sync_copy(ie_hbm, ie_b)
        pltpu.sync_copy(pe_hbm, pe_b)

        wi = wi_b[wid, pl.ds(0, _LANES)]
        base = wi[0]
        nch = wi[1]
        a0 = wi[2]
        ac0 = wi[3]

        bufs = ((ib0, ob0, si0, so0), (ib1, ob1, si1, so1))

        def start_in(k, ib, si):
            off = pl.multiple_of(base + k * CH, 8)
            pltpu.async_copy(x_hbm.at[pl.ds(off, CH)], ib, si)

        def start_out(k, ob, so):
            off = pl.multiple_of(base + k * CH, 8)
            pltpu.async_copy(ob, out_hbm.at[pl.ds(off, CH)], so)

        def wait_in(ib, si):
            pltpu.make_async_copy(x_hbm.at[pl.ds(0, CH)], ib, si).wait()

        def wait_out(ob, so):
            pltpu.make_async_copy(ob, out_hbm.at[pl.ds(0, CH)], so).wait()

        start_in(0, ib0, si0)
        start_in(1, ib1, si1)

        def compute(ib, ob, a, poff):
            sidx = idx_b[a, pl.ds(0, _LANES)][0]
            ie_vecs = [ie_b[sidx, pl.ds(j * _LANES, _LANES)] for j in range(NV)]

            @plsc.parallel_loop(0, CH, unroll=2)
            def row(r):
                for j in range(NV):
                    sl = pl.ds(j * _LANES, _LANES)
                    ob[r, sl] = ib[r, sl] + pe_b[poff + r, sl] + ie_vecs[j]

        def step(k, carry):
            a, ac, poff = carry

            def do(ib, ob, si, so):
                wait_in(ib, si)

                @pl.when(k > 1)
                def _():
                    wait_out(ob, so)

                compute(ib, ob, a, poff)
                start_out(k, ob, so)

                @pl.when(k + 2 < nch)
                def _():
                    start_in(k + 2, ib, si)

            @pl.when(jnp.bitwise_and(k, 1) == 0)
            def _():
                do(*bufs[0])

            @pl.when(jnp.bitwise_and(k, 1) == 1)
            def _():
                do(*bufs[1])

            poff = jnp.where(poff + CH == T, 0, poff + CH)
            ac = ac + 1
            bump = ac == CPA
            a = jnp.where(bump, a + 1, a)
            ac = jnp.where(bump, 0, ac)
            return (a, ac, poff)

        lax.fori_loop(0, nch, step, (a0, ac0, jnp.int32(0)))

        wait_out(ob0, so0)
        wait_out(ob1, so1)

    return sc_add


def _worker_init(R, T, P, NW=32, CH=40):
    nblk = R // T
    hi = nblk // NW + 1
    lo = nblk // NW
    nhi = nblk - lo * NW
    cpb = T // CH
    cpa = (P * T) // CH
    rows = []
    sb = 0
    for w in range(NW):
        nb = hi if w < nhi else lo
        g0 = sb * cpb
        rows.append([sb * T, nb * cpb, g0 // cpa, g0 % cpa] + [0] * 12)
        sb += nb
    return np.asarray(rows, dtype=np.int32)


def kernel(x, pe, ie, id_enc_shuffle, num_a, num_p, num_t, t_offset):
    B, N, D = x.shape
    A = id_enc_shuffle.shape[0]
    T = pe.shape[1]
    P = N // (A * T)
    AP = 32

    x2 = x.reshape(N, D)
    pe2 = pe.reshape(T, D)
    ie2 = jnp.zeros((AP, D), jnp.float32).at[: ie.shape[1]].set(ie.reshape(ie.shape[1], D))
    idxp = jnp.zeros((AP, 16), jnp.int32).at[:A].set(
        jnp.broadcast_to(id_enc_shuffle.astype(jnp.int32)[:, None], (A, 16)))
    winit = jnp.asarray(_worker_init(N, T, P))

    out2 = _build_sc_call(N, D, T, A, P, AP)(x2, pe2, ie2, idxp, winit)
    return out2.reshape(B, N, D)


# speedup vs baseline: 4.2711x; 1.0122x over previous
import functools

import jax
import jax.numpy as jnp
import numpy as np
from jax import lax
from jax.experimental import pallas as pl
from jax.experimental.pallas import tpu as pltpu
from jax.experimental.pallas import tpu_sc as plsc

_LANES = 16


def _build_sc_call(R, D, T, A, P, AP):
    NC, NS = 2, 16
    NW = NC * NS
    CH = 40
    CPB = T // CH
    CPA = (P * T) // CH
    NV = D // _LANES

    NBLK = R // T
    assert T % CH == 0 and (P * T) % CH == 0 and CH % 8 == 0

    mesh = plsc.VectorSubcoreMesh(core_axis_name="c", subcore_axis_name="s")

    @functools.partial(
        pl.kernel,
        out_type=jax.ShapeDtypeStruct((R, D), jnp.float32),
        mesh=mesh,
        scratch_types=[
            pltpu.VMEM((T, D), jnp.float32),
            pltpu.VMEM((AP, D), jnp.float32),
            pltpu.VMEM((AP, _LANES), jnp.int32),
            pltpu.VMEM((NW, _LANES), jnp.int32),
            pltpu.VMEM((CH, D), jnp.float32),
            pltpu.VMEM((CH, D), jnp.float32),
            pltpu.VMEM((CH, D), jnp.float32),
            pltpu.VMEM((CH, D), jnp.float32),
            pltpu.SemaphoreType.DMA,
            pltpu.SemaphoreType.DMA,
            pltpu.SemaphoreType.DMA,
            pltpu.SemaphoreType.DMA,
        ],
    )
    def sc_add(x_hbm, pe_hbm, ie_hbm, idx_hbm, winit_hbm, out_hbm,
               pe_b, ie_b, idx_b, wi_b, ib0, ib1, ob0, ob1,
               si0, si1, so0, so1):
        wid = lax.axis_index("s") * NC + lax.axis_index("c")

        pltpu.sync_copy(winit_hbm, wi_b)

        wi = wi_b[wid, pl.ds(0, _LANES)]
        base = wi[0]
        nch = wi[1]
        a0 = wi[2]
        ac0 = wi[3]
        poff0 = wi[4]

        bufs = ((ib0, ob0, si0, so0), (ib1, ob1, si1, so1))

        def start_in(k, ib, si):
            off = pl.multiple_of(base + k * CH, 8)
            pltpu.async_copy(x_hbm.at[pl.ds(off, CH)], ib, si)

        def start_out(k, ob, so):
            off = pl.multiple_of(base + k * CH, 8)
            pltpu.async_copy(ob, out_hbm.at[pl.ds(off, CH)], so)

        def wait_in(ib, si):
            pltpu.make_async_copy(x_hbm.at[pl.ds(0, CH)], ib, si).wait()

        def wait_out(ob, so):
            pltpu.make_async_copy(ob, out_hbm.at[pl.ds(0, CH)], so).wait()

        start_in(0, ib0, si0)
        start_in(1, ib1, si1)

        pltpu.sync_copy(idx_hbm, idx_b)
        pltpu.sync_copy(ie_hbm, ie_b)
        pltpu.sync_copy(pe_hbm, pe_b)

        def compute(ib, ob, a, poff):
            sidx = idx_b[a, pl.ds(0, _LANES)][0]
            ie_vecs = [ie_b[sidx, pl.ds(j * _LANES, _LANES)] for j in range(NV)]

            @plsc.parallel_loop(0, CH, unroll=2)
            def row(r):
                for j in range(NV):
                    sl = pl.ds(j * _LANES, _LANES)
                    ob[r, sl] = ib[r, sl] + pe_b[poff + r, sl] + ie_vecs[j]

        def step(k, carry):
            a, ac, poff = carry

            def do(ib, ob, si, so):
                wait_in(ib, si)

                @pl.when(k > 1)
                def _():
                    wait_out(ob, so)

                compute(ib, ob, a, poff)
                start_out(k, ob, so)

                @pl.when(k + 2 < nch)
                def _():
                    start_in(k + 2, ib, si)

            @pl.when(jnp.bitwise_and(k, 1) == 0)
            def _():
                do(*bufs[0])

            @pl.when(jnp.bitwise_and(k, 1) == 1)
            def _():
                do(*bufs[1])

            poff = jnp.where(poff + CH == T, 0, poff + CH)
            ac = ac + 1
            bump = ac == CPA
            a = jnp.where(bump, a + 1, a)
            ac = jnp.where(bump, 0, ac)
            return (a, ac, poff)

        lax.fori_loop(0, nch, step, (a0, ac0, poff0))

        wait_out(ob0, so0)
        wait_out(ob1, so1)

    return sc_add


def _worker_init(R, T, P, NW=32, CH=40):
    nchunks = R // CH
    cpb = T // CH
    cpa = (P * T) // CH
    rows = []
    g0 = 0
    for w in range(NW):
        nc = nchunks // NW + (1 if w < nchunks % NW else 0)
        rows.append([g0 * CH, nc, g0 // cpa, g0 % cpa, (g0 % cpb) * CH] + [0] * 11)
        g0 += nc
    return np.asarray(rows, dtype=np.int32)


def kernel(x, pe, ie, id_enc_shuffle, num_a, num_p, num_t, t_offset):
    B, N, D = x.shape
    A = id_enc_shuffle.shape[0]
    T = pe.shape[1]
    P = N // (A * T)
    AP = 32

    x2 = x.reshape(N, D)
    pe2 = pe.reshape(T, D)
    ie2 = jnp.zeros((AP, D), jnp.float32).at[: ie.shape[1]].set(ie.reshape(ie.shape[1], D))
    idxp = jnp.zeros((AP, 16), jnp.int32).at[:A].set(
        jnp.broadcast_to(id_enc_shuffle.astype(jnp.int32)[:, None], (A, 16)))
    winit = jnp.asarray(_worker_init(N, T, P))

    out2 = _build_sc_call(N, D, T, A, P, AP)(x2, pe2, ie2, idxp, winit)
    return out2.reshape(B, N, D)
